# baseline (speedup 1.0000x reference)
import functools

import jax
import jax.numpy as jnp
from jax import lax
from jax.experimental import pallas as pl
from jax.experimental.pallas import tpu as pltpu
from jax.experimental.pallas import tpu_sc as plsc

_INTERPRET = False

_B, _D, _T = 16, 256, 1024
_M = _B * _T
_K = 8192

_TM = 512
_TN = 2048
_NI = _M // _TM
_NJ = _K // _TN


_CH = 32


def _argmin_body(wn_ref, znt_ref, codes_ref, s_ref, best_d_ref, best_i_ref):
    j = pl.program_id(1)
    s_ref[...] = jnp.dot(wn_ref[...], znt_ref[...],
                         preferred_element_type=jnp.float32)
    big = jnp.int32(2**31 - 1)
    iota = lax.broadcasted_iota(jnp.int32, (_CH, _TM), 0)

    def chunk(c, carry):
        run_d, run_i = carry
        dc = 1.0 - s_ref[pl.ds(c * _CH, _CH), :]
        rows = iota + (c * _CH + j * _TN)
        take = dc < run_d
        return jnp.where(take, dc, run_d), jnp.where(take, rows, run_i)

    run_d, run_i = lax.fori_loop(
        0, _TN // _CH, chunk,
        (jnp.full((_CH, _TM), jnp.inf, jnp.float32),
         jnp.full((_CH, _TM), big, jnp.int32)))
    m = jnp.min(run_d, axis=0)
    idx = jnp.min(jnp.where(run_d == m[None, :], run_i, big), axis=0)

    @pl.when(j == 0)
    def _():
        best_d_ref[0, :] = m
        best_i_ref[0, :] = idx

    @pl.when(j > 0)
    def _():
        take = m < best_d_ref[0, :]
        best_d_ref[0, :] = jnp.where(take, m, best_d_ref[0, :])
        best_i_ref[0, :] = jnp.where(take, idx, best_i_ref[0, :])

    @pl.when(j == _NJ - 1)
    def _():
        codes_ref[0, 0, :] = best_i_ref[0, :]


def _argmin_codes(wn, znt):
    out = pl.pallas_call(
        _argmin_body,
        grid=(_NI, _NJ),
        in_specs=[
            pl.BlockSpec((_TN, _D), lambda i, j: (j, 0)),
            pl.BlockSpec((_D, _TM), lambda i, j: (0, i)),
        ],
        out_specs=pl.BlockSpec((1, 1, _TM), lambda i, j: (i, 0, 0)),
        out_shape=jax.ShapeDtypeStruct((_NI, 1, _TM), jnp.int32),
        scratch_shapes=[
            pltpu.VMEM((_TN, _TM), jnp.float32),
            pltpu.VMEM((1, _TM), jnp.float32),
            pltpu.VMEM((1, _TM), jnp.int32),
        ],
        compiler_params=pltpu.CompilerParams(
            dimension_semantics=("parallel", "arbitrary")),
        interpret=_INTERPRET,
    )(wn, znt)
    return out.reshape(_M)


def _loss_body(q_ref, z_ref, out_ref, acc_ref):
    i = pl.program_id(0)

    @pl.when(i == 0)
    def _():
        acc_ref[0, 0] = 0.0

    dlt = q_ref[...] - z_ref[...]
    acc_ref[0, 0] += jnp.sum(dlt * dlt)

    @pl.when(i == pl.num_programs(0) - 1)
    def _():
        out_ref[...] = jnp.full((1, 1), acc_ref[0, 0] * (0.25 / (_M * _D)),
                                jnp.float32)


def _loss(q, zp):
    out = pl.pallas_call(
        _loss_body,
        grid=(_NI,),
        in_specs=[
            pl.BlockSpec((_TM, _D), lambda i: (i, 0)),
            pl.BlockSpec((_TM, _D), lambda i: (i, 0)),
        ],
        out_specs=pl.BlockSpec((1, 1), lambda i: (0, 0)),
        out_shape=jax.ShapeDtypeStruct((1, 1), jnp.float32),
        scratch_shapes=[pltpu.SMEM((1, 1), jnp.float32)],
        interpret=_INTERPRET,
    )(q, zp)
    return out[0, 0]


def _sc_gather(codebook, codes):
    info = plsc.get_sparse_core_info()
    nc, ns = info.num_cores, info.num_subcores
    nw = nc * ns
    b_per_w = _M // nw
    chunk = 128
    n_chunks = b_per_w // chunk
    mesh = plsc.VectorSubcoreMesh(core_axis_name="c", subcore_axis_name="s")

    @functools.partial(
        pl.kernel,
        mesh=mesh,
        out_type=jax.ShapeDtypeStruct((_M, _D), jnp.float32),
        scratch_types=[
            pltpu.VMEM((chunk,), jnp.int32),
            pltpu.VMEM((chunk, _D), jnp.float32),
            pltpu.SemaphoreType.DMA,
        ],
    )
    def k(cb_hbm, idx_hbm, out_hbm, idx_v, rows_v, sem):
        wid = lax.axis_index("s") * nc + lax.axis_index("c")
        base = wid * b_per_w

        def body(g, carry):
            off = base + g * chunk
            pltpu.sync_copy(idx_hbm.at[pl.ds(off, chunk)], idx_v)
            pltpu.async_copy(cb_hbm.at[idx_v], rows_v, sem).wait()
            pltpu.sync_copy(rows_v, out_hbm.at[pl.ds(off, chunk)])
            return carry

        lax.fori_loop(0, n_chunks, body, 0)

    return k(codebook, codes)


def kernel(z, codebook):
    b, d, t = z.shape
    zp = jnp.transpose(z, (0, 2, 1)).reshape(-1, d)
    zn = zp / jnp.maximum(jnp.linalg.norm(zp, axis=-1, keepdims=True), 1e-6)
    wn = codebook / jnp.maximum(
        jnp.linalg.norm(codebook, axis=-1, keepdims=True), 1e-6)
    codes = _argmin_codes(wn, zn.T)
    q = _sc_gather(codebook, codes)
    loss = _loss(q, zp)
    q_out = jnp.transpose(q.reshape(b, t, d), (0, 2, 1))
    return q_out, loss, codes.reshape(b, t)


# speedup vs baseline: 1.4016x; 1.4016x over previous
import functools

import jax
import jax.numpy as jnp
from jax import lax
from jax.experimental import pallas as pl
from jax.experimental.pallas import tpu as pltpu
from jax.experimental.pallas import tpu_sc as plsc

_INTERPRET = False

_B, _D, _T = 16, 256, 1024
_M = _B * _T
_K = 8192

_TM = 512
_TN = 2048
_NI = _M // _TM
_NJ = _K // _TN


_NSUB = 4


def _argmin_body(wn_ref, znt_ref, codes_ref, best_d_ref, best_i_ref):
    j = pl.program_id(1)
    big = jnp.int32(2**31 - 1)
    rows_per = _TN // _NSUB
    acc_d = jnp.full((8, _TM), jnp.inf, jnp.float32)
    acc_g = jnp.full((8, _TM), big, jnp.int32)
    for c in range(_NSUB):
        s_c = jnp.dot(wn_ref[pl.ds(c * rows_per, rows_per), :], znt_ref[...],
                      preferred_element_type=jnp.float32)
        for r in range(rows_per // 8):
            dblk = 1.0 - s_c[r * 8:(r + 1) * 8, :]
            g = jnp.full((8, _TM), c * rows_per // 8 + r, jnp.int32)
            take = dblk < acc_d
            acc_d = jnp.where(take, dblk, acc_d)
            acc_g = jnp.where(take, g, acc_g)
    sub = lax.broadcasted_iota(jnp.int32, (8, _TM), 0)
    acc_i = acc_g * 8 + sub + j * _TN
    m = jnp.min(acc_d, axis=0)
    idx = jnp.min(jnp.where(acc_d == m[None, :], acc_i, big), axis=0)

    @pl.when(j == 0)
    def _():
        best_d_ref[0, :] = m
        best_i_ref[0, :] = idx

    @pl.when(j > 0)
    def _():
        take = m < best_d_ref[0, :]
        best_d_ref[0, :] = jnp.where(take, m, best_d_ref[0, :])
        best_i_ref[0, :] = jnp.where(take, idx, best_i_ref[0, :])

    @pl.when(j == _NJ - 1)
    def _():
        codes_ref[0, 0, :] = best_i_ref[0, :]


def _argmin_codes(wn, znt):
    out = pl.pallas_call(
        _argmin_body,
        grid=(_NI, _NJ),
        in_specs=[
            pl.BlockSpec((_TN, _D), lambda i, j: (j, 0)),
            pl.BlockSpec((_D, _TM), lambda i, j: (0, i)),
        ],
        out_specs=pl.BlockSpec((1, 1, _TM), lambda i, j: (i, 0, 0)),
        out_shape=jax.ShapeDtypeStruct((_NI, 1, _TM), jnp.int32),
        scratch_shapes=[
            pltpu.VMEM((1, _TM), jnp.float32),
            pltpu.VMEM((1, _TM), jnp.int32),
        ],
        compiler_params=pltpu.CompilerParams(
            dimension_semantics=("parallel", "arbitrary")),
        interpret=_INTERPRET,
    )(wn, znt)
    return out.reshape(_M)


def _loss_body(q_ref, z_ref, out_ref, acc_ref):
    i = pl.program_id(0)

    @pl.when(i == 0)
    def _():
        acc_ref[0, 0] = 0.0

    dlt = q_ref[...] - z_ref[...]
    acc_ref[0, 0] += jnp.sum(dlt * dlt)

    @pl.when(i == pl.num_programs(0) - 1)
    def _():
        out_ref[...] = jnp.full((1, 1), acc_ref[0, 0] * (0.25 / (_M * _D)),
                                jnp.float32)


def _loss(q, zp):
    out = pl.pallas_call(
        _loss_body,
        grid=(_NI,),
        in_specs=[
            pl.BlockSpec((_TM, _D), lambda i: (i, 0)),
            pl.BlockSpec((_TM, _D), lambda i: (i, 0)),
        ],
        out_specs=pl.BlockSpec((1, 1), lambda i: (0, 0)),
        out_shape=jax.ShapeDtypeStruct((1, 1), jnp.float32),
        scratch_shapes=[pltpu.SMEM((1, 1), jnp.float32)],
        interpret=_INTERPRET,
    )(q, zp)
    return out[0, 0]


def _sc_gather(codebook, codes):
    info = plsc.get_sparse_core_info()
    nc, ns = info.num_cores, info.num_subcores
    nw = nc * ns
    b_per_w = _M // nw
    chunk = 128
    n_chunks = b_per_w // chunk
    mesh = plsc.VectorSubcoreMesh(core_axis_name="c", subcore_axis_name="s")

    @functools.partial(
        pl.kernel,
        mesh=mesh,
        out_type=jax.ShapeDtypeStruct((_M, _D), jnp.float32),
        scratch_types=[
            pltpu.VMEM((chunk,), jnp.int32),
            pltpu.VMEM((chunk, _D), jnp.float32),
            pltpu.SemaphoreType.DMA,
        ],
    )
    def k(cb_hbm, idx_hbm, out_hbm, idx_v, rows_v, sem):
        wid = lax.axis_index("s") * nc + lax.axis_index("c")
        base = wid * b_per_w

        def body(g, carry):
            off = base + g * chunk
            pltpu.sync_copy(idx_hbm.at[pl.ds(off, chunk)], idx_v)
            pltpu.async_copy(cb_hbm.at[idx_v], rows_v, sem).wait()
            pltpu.sync_copy(rows_v, out_hbm.at[pl.ds(off, chunk)])
            return carry

        lax.fori_loop(0, n_chunks, body, 0)

    return k(codebook, codes)


def kernel(z, codebook):
    b, d, t = z.shape
    zp = jnp.transpose(z, (0, 2, 1)).reshape(-1, d)
    zn = zp / jnp.maximum(jnp.linalg.norm(zp, axis=-1, keepdims=True), 1e-6)
    wn = codebook / jnp.maximum(
        jnp.linalg.norm(codebook, axis=-1, keepdims=True), 1e-6)
    codes = _argmin_codes(wn, zn.T)
    q = _sc_gather(codebook, codes)
    loss = _loss(q, zp)
    q_out = jnp.transpose(q.reshape(b, t, d), (0, 2, 1))
    return q_out, loss, codes.reshape(b, t)


# speedup vs baseline: 1.7188x; 1.2263x over previous
import functools

import jax
import jax.numpy as jnp
from jax import lax
from jax.experimental import pallas as pl
from jax.experimental.pallas import tpu as pltpu
from jax.experimental.pallas import tpu_sc as plsc

_INTERPRET = False

_B, _D, _T = 16, 256, 1024
_M = _B * _T
_K = 8192

_TM = 512
_TN = 2048
_NI = _M // _TM
_NJ = _K // _TN


_NSUB = 4


def _argmin_body(wn_ref, znt_ref, codes_ref, best_d_ref, best_i_ref):
    j = pl.program_id(0)
    i = pl.program_id(1)
    big = jnp.int32(2**31 - 1)
    rows_per = _TN // _NSUB
    acc_d = jnp.full((8, _TM), jnp.inf, jnp.float32)
    acc_g = jnp.full((8, _TM), big, jnp.int32)
    for c in range(_NSUB):
        s_c = jnp.dot(wn_ref[pl.ds(c * rows_per, rows_per), :], znt_ref[...],
                      preferred_element_type=jnp.float32)
        for r in range(rows_per // 8):
            dblk = 1.0 - s_c[r * 8:(r + 1) * 8, :]
            g = jnp.full((8, _TM), c * rows_per // 8 + r, jnp.int32)
            take = dblk < acc_d
            acc_d = jnp.where(take, dblk, acc_d)
            acc_g = jnp.where(take, g, acc_g)
    sub = lax.broadcasted_iota(jnp.int32, (8, _TM), 0)
    acc_i = acc_g * 8 + sub + j * _TN
    m = jnp.min(acc_d, axis=0)
    idx = jnp.min(jnp.where(acc_d == m[None, :], acc_i, big), axis=0)

    @pl.when(j == 0)
    def _():
        best_d_ref[pl.ds(i, 1), :] = m[None, :]
        best_i_ref[pl.ds(i, 1), :] = idx[None, :]

    @pl.when(j > 0)
    def _():
        take = m[None, :] < best_d_ref[pl.ds(i, 1), :]
        best_d_ref[pl.ds(i, 1), :] = jnp.where(
            take, m[None, :], best_d_ref[pl.ds(i, 1), :])
        best_i_ref[pl.ds(i, 1), :] = jnp.where(
            take, idx[None, :], best_i_ref[pl.ds(i, 1), :])

    codes_ref[0, 0, :] = best_i_ref[pl.ds(i, 1), :][0, :]


def _argmin_codes(wn, znt):
    out = pl.pallas_call(
        _argmin_body,
        grid=(_NJ, _NI),
        in_specs=[
            pl.BlockSpec((_TN, _D), lambda j, i: (j, 0)),
            pl.BlockSpec((_D, _TM), lambda j, i: (0, i)),
        ],
        out_specs=pl.BlockSpec((1, 1, _TM), lambda j, i: (i, 0, 0)),
        out_shape=jax.ShapeDtypeStruct((_NI, 1, _TM), jnp.int32),
        scratch_shapes=[
            pltpu.VMEM((_NI, _TM), jnp.float32),
            pltpu.VMEM((_NI, _TM), jnp.int32),
        ],
        compiler_params=pltpu.CompilerParams(
            dimension_semantics=("arbitrary", "arbitrary")),
        interpret=_INTERPRET,
    )(wn, znt)
    return out.reshape(_M)


def _loss_body(q_ref, z_ref, qt_ref, loss_ref, acc_ref):
    i = pl.program_id(0)

    @pl.when(i == 0)
    def _():
        acc_ref[0, 0] = 0.0

    qt = jnp.transpose(q_ref[...], (1, 0))
    qt_ref[0] = qt
    dlt = qt - z_ref[0]
    acc_ref[0, 0] += jnp.sum(dlt * dlt)

    @pl.when(i == pl.num_programs(0) - 1)
    def _():
        loss_ref[...] = jnp.full((1, 1), acc_ref[0, 0] * (0.25 / (_M * _D)),
                                 jnp.float32)


def _loss_and_qout(q, z):
    qt, loss = pl.pallas_call(
        _loss_body,
        grid=(_NI,),
        in_specs=[
            pl.BlockSpec((_TM, _D), lambda i: (i, 0)),
            pl.BlockSpec((1, _D, _TM), lambda i: (i // (_T // _TM), 0,
                                                  i % (_T // _TM))),
        ],
        out_specs=[
            pl.BlockSpec((1, _D, _TM), lambda i: (i // (_T // _TM), 0,
                                                  i % (_T // _TM))),
            pl.BlockSpec((1, 1), lambda i: (0, 0)),
        ],
        out_shape=[
            jax.ShapeDtypeStruct((_B, _D, _T), jnp.float32),
            jax.ShapeDtypeStruct((1, 1), jnp.float32),
        ],
        scratch_shapes=[pltpu.SMEM((1, 1), jnp.float32)],
        interpret=_INTERPRET,
    )(q, z)
    return qt, loss[0, 0]


def _sc_gather(codebook, codes):
    info = plsc.get_sparse_core_info()
    nc, ns = info.num_cores, info.num_subcores
    nw = nc * ns
    b_per_w = _M // nw
    chunk = 128
    n_chunks = b_per_w // chunk
    mesh = plsc.VectorSubcoreMesh(core_axis_name="c", subcore_axis_name="s")

    @functools.partial(
        pl.kernel,
        mesh=mesh,
        out_type=jax.ShapeDtypeStruct((_M, _D), jnp.float32),
        scratch_types=[
            pltpu.VMEM((chunk,), jnp.int32),
            pltpu.VMEM((chunk, _D), jnp.float32),
            pltpu.SemaphoreType.DMA,
        ],
    )
    def k(cb_hbm, idx_hbm, out_hbm, idx_v, rows_v, sem):
        wid = lax.axis_index("s") * nc + lax.axis_index("c")
        base = wid * b_per_w

        def body(g, carry):
            off = base + g * chunk
            pltpu.sync_copy(idx_hbm.at[pl.ds(off, chunk)], idx_v)
            pltpu.async_copy(cb_hbm.at[idx_v], rows_v, sem).wait()
            pltpu.sync_copy(rows_v, out_hbm.at[pl.ds(off, chunk)])
            return carry

        lax.fori_loop(0, n_chunks, body, 0)

    return k(codebook, codes)


def kernel(z, codebook):
    b, d, t = z.shape
    zp = jnp.transpose(z, (0, 2, 1)).reshape(-1, d)
    zn = zp / jnp.maximum(jnp.linalg.norm(zp, axis=-1, keepdims=True), 1e-6)
    wn = codebook / jnp.maximum(
        jnp.linalg.norm(codebook, axis=-1, keepdims=True), 1e-6)
    codes = _argmin_codes(wn, zn.T)
    q = _sc_gather(codebook, codes)
    q_out, loss = _loss_and_qout(q, z)
    return q_out, loss, codes.reshape(b, t)


# speedup vs baseline: 2.2266x; 1.2954x over previous
import functools

import jax
import jax.numpy as jnp
from jax import lax
from jax.experimental import pallas as pl
from jax.experimental.pallas import tpu as pltpu
from jax.experimental.pallas import tpu_sc as plsc

_INTERPRET = False

_B, _D, _T = 16, 256, 1024
_M = _B * _T
_K = 8192

_TM = 1024
_TN = 2048
_NI = _M // _TM
_NJ = _K // _TN

_NSUB = 4


def _argmin_body(cb_ref, nw_ref, z_ref, nz_ref, codes_ref,
                 wn_ref, best_d_ref, best_i_ref):
    j = pl.program_id(0)
    i = pl.program_id(1)

    @pl.when(i == 0)
    def _():
        wn_ref[...] = cb_ref[...] / jnp.maximum(nw_ref[...], 1e-6)

    znt = z_ref[0] / jnp.maximum(nz_ref[0], 1e-6)

    big = jnp.int32(2**31 - 1)
    rows_per = _TN // _NSUB
    acc_d = jnp.full((8, _TM), jnp.inf, jnp.float32)
    acc_g = jnp.full((8, _TM), big, jnp.int32)
    for c in range(_NSUB):
        s_c = jnp.dot(wn_ref[pl.ds(c * rows_per, rows_per), :], znt,
                      preferred_element_type=jnp.float32)
        for r in range(rows_per // 8):
            dblk = 1.0 - s_c[r * 8:(r + 1) * 8, :]
            g = jnp.full((8, _TM), c * rows_per // 8 + r, jnp.int32)
            take = dblk < acc_d
            acc_d = jnp.where(take, dblk, acc_d)
            acc_g = jnp.where(take, g, acc_g)
    sub = lax.broadcasted_iota(jnp.int32, (8, _TM), 0)
    acc_i = acc_g * 8 + sub + j * _TN
    m = jnp.min(acc_d, axis=0)
    idx = jnp.min(jnp.where(acc_d == m[None, :], acc_i, big), axis=0)

    @pl.when(j == 0)
    def _():
        best_d_ref[pl.ds(i, 1), :] = m[None, :]
        best_i_ref[pl.ds(i, 1), :] = idx[None, :]

    @pl.when(j > 0)
    def _():
        take = m[None, :] < best_d_ref[pl.ds(i, 1), :]
        best_d_ref[pl.ds(i, 1), :] = jnp.where(
            take, m[None, :], best_d_ref[pl.ds(i, 1), :])
        best_i_ref[pl.ds(i, 1), :] = jnp.where(
            take, idx[None, :], best_i_ref[pl.ds(i, 1), :])

    codes_ref[0, 0, :] = best_i_ref[pl.ds(i, 1), :][0, :]


def _argmin_codes(codebook, nw, z, nz):
    out = pl.pallas_call(
        _argmin_body,
        grid=(_NJ, _NI),
        in_specs=[
            pl.BlockSpec((_TN, _D), lambda j, i: (j, 0)),
            pl.BlockSpec((_TN, 1), lambda j, i: (j, 0)),
            pl.BlockSpec((1, _D, _TM), lambda j, i: (i, 0, 0)),
            pl.BlockSpec((1, 1, _TM), lambda j, i: (i, 0, 0)),
        ],
        out_specs=pl.BlockSpec((1, 1, _TM), lambda j, i: (i, 0, 0)),
        out_shape=jax.ShapeDtypeStruct((_NI, 1, _TM), jnp.int32),
        scratch_shapes=[
            pltpu.VMEM((_TN, _D), jnp.float32),
            pltpu.VMEM((_NI, _TM), jnp.float32),
            pltpu.VMEM((_NI, _TM), jnp.int32),
        ],
        compiler_params=pltpu.CompilerParams(
            dimension_semantics=("arbitrary", "arbitrary")),
        interpret=_INTERPRET,
    )(codebook, nw, z.reshape(_NI, _D, _TM), nz)
    return out.reshape(_M)


def _loss_body(q_ref, z_ref, qt_ref, loss_ref, acc_ref):
    i = pl.program_id(0)

    @pl.when(i == 0)
    def _():
        acc_ref[0, 0] = 0.0

    qt = jnp.transpose(q_ref[...], (1, 0))
    qt_ref[0] = qt
    dlt = qt - z_ref[0]
    acc_ref[0, 0] += jnp.sum(dlt * dlt)

    @pl.when(i == pl.num_programs(0) - 1)
    def _():
        loss_ref[...] = jnp.full((1, 1), acc_ref[0, 0] * (0.25 / (_M * _D)),
                                 jnp.float32)


_TL = 512


def _loss_and_qout(q, z):
    nblk = _M // _TL
    qt, loss = pl.pallas_call(
        _loss_body,
        grid=(nblk,),
        in_specs=[
            pl.BlockSpec((_TL, _D), lambda i: (i, 0)),
            pl.BlockSpec((1, _D, _TL), lambda i: (i // (_T // _TL), 0,
                                                  i % (_T // _TL))),
        ],
        out_specs=[
            pl.BlockSpec((1, _D, _TL), lambda i: (i // (_T // _TL), 0,
                                                  i % (_T // _TL))),
            pl.BlockSpec((1, 1), lambda i: (0, 0)),
        ],
        out_shape=[
            jax.ShapeDtypeStruct((_B, _D, _T), jnp.float32),
            jax.ShapeDtypeStruct((1, 1), jnp.float32),
        ],
        scratch_shapes=[pltpu.SMEM((1, 1), jnp.float32)],
        interpret=_INTERPRET,
    )(q, z)
    return qt, loss[0, 0]


def _sc_gather(codebook, codes):
    info = plsc.get_sparse_core_info()
    nc, ns = info.num_cores, info.num_subcores
    nw = nc * ns
    b_per_w = _M // nw
    chunk = 128
    n_chunks = b_per_w // chunk
    mesh = plsc.VectorSubcoreMesh(core_axis_name="c", subcore_axis_name="s")

    @functools.partial(
        pl.kernel,
        mesh=mesh,
        out_type=jax.ShapeDtypeStruct((_M, _D), jnp.float32),
        scratch_types=[
            pltpu.VMEM((chunk,), jnp.int32),
            pltpu.VMEM((chunk, _D), jnp.float32),
            pltpu.SemaphoreType.DMA,
        ],
    )
    def k(cb_hbm, idx_hbm, out_hbm, idx_v, rows_v, sem):
        wid = lax.axis_index("s") * nc + lax.axis_index("c")
        base = wid * b_per_w

        def body(g, carry):
            off = base + g * chunk
            pltpu.sync_copy(idx_hbm.at[pl.ds(off, chunk)], idx_v)
            pltpu.async_copy(cb_hbm.at[idx_v], rows_v, sem).wait()
            pltpu.sync_copy(rows_v, out_hbm.at[pl.ds(off, chunk)])
            return carry

        lax.fori_loop(0, n_chunks, body, 0)

    return k(codebook, codes)


def kernel(z, codebook):
    b, d, t = z.shape
    zp = jnp.transpose(z, (0, 2, 1)).reshape(-1, d)
    nz = jnp.linalg.norm(zp, axis=-1)
    nw = jnp.linalg.norm(codebook, axis=-1, keepdims=True)
    codes = _argmin_codes(codebook, nw, z, nz.reshape(_NI, 1, _TM))
    q = _sc_gather(codebook, codes)
    q_out, loss = _loss_and_qout(q, z)
    return q_out, loss, codes.reshape(b, t)


# speedup vs baseline: 2.4280x; 1.0905x over previous
import functools

import jax
import jax.numpy as jnp
from jax import lax
from jax.experimental import pallas as pl
from jax.experimental.pallas import tpu as pltpu
from jax.experimental.pallas import tpu_sc as plsc

_INTERPRET = False

_B, _D, _T = 16, 256, 1024
_M = _B * _T
_K = 8192

_TM = 1024
_TN = 2048
_NI = _M // _TM
_NJ = _K // _TN

_NSUB = 4


def _argmin_body(cb_ref, nw_ref, z_ref, nz_ref, codes_ref,
                 wn_ref, best_d_ref, best_i_ref):
    j = pl.program_id(0)
    i = pl.program_id(1)

    @pl.when(i == 0)
    def _():
        wn_ref[...] = cb_ref[...] / jnp.maximum(nw_ref[...], 1e-6)

    znt = z_ref[0] / jnp.maximum(nz_ref[0], 1e-6)

    big = jnp.int32(2**31 - 1)
    rows_per = _TN // _NSUB
    acc_d = jnp.full((8, _TM), jnp.inf, jnp.float32)
    acc_g = jnp.full((8, _TM), big, jnp.int32)
    for c in range(_NSUB):
        s_c = jnp.dot(wn_ref[pl.ds(c * rows_per, rows_per), :], znt,
                      preferred_element_type=jnp.float32)
        for r in range(rows_per // 8):
            dblk = 1.0 - s_c[r * 8:(r + 1) * 8, :]
            g = jnp.full((8, _TM), c * rows_per // 8 + r, jnp.int32)
            take = dblk < acc_d
            acc_d = jnp.where(take, dblk, acc_d)
            acc_g = jnp.where(take, g, acc_g)
    sub = lax.broadcasted_iota(jnp.int32, (8, _TM), 0)
    acc_i = acc_g * 8 + sub + j * _TN
    m = jnp.min(acc_d, axis=0)
    idx = jnp.min(jnp.where(acc_d == m[None, :], acc_i, big), axis=0)

    @pl.when(j == 0)
    def _():
        best_d_ref[pl.ds(i, 1), :] = m[None, :]
        best_i_ref[pl.ds(i, 1), :] = idx[None, :]

    @pl.when(j > 0)
    def _():
        take = m[None, :] < best_d_ref[pl.ds(i, 1), :]
        best_d_ref[pl.ds(i, 1), :] = jnp.where(
            take, m[None, :], best_d_ref[pl.ds(i, 1), :])
        best_i_ref[pl.ds(i, 1), :] = jnp.where(
            take, idx[None, :], best_i_ref[pl.ds(i, 1), :])

    codes_ref[0, 0, :] = best_i_ref[pl.ds(i, 1), :][0, :]


def _argmin_codes(codebook, nw, z, nz):
    out = pl.pallas_call(
        _argmin_body,
        grid=(_NJ, _NI),
        in_specs=[
            pl.BlockSpec((_TN, _D), lambda j, i: (j, 0)),
            pl.BlockSpec((_TN, 1), lambda j, i: (j, 0)),
            pl.BlockSpec((1, _D, _TM), lambda j, i: (i, 0, 0)),
            pl.BlockSpec((1, 1, _TM), lambda j, i: (i, 0, 0)),
        ],
        out_specs=pl.BlockSpec((1, 1, _TM), lambda j, i: (i, 0, 0)),
        out_shape=jax.ShapeDtypeStruct((_NI, 1, _TM), jnp.int32),
        scratch_shapes=[
            pltpu.VMEM((_TN, _D), jnp.float32),
            pltpu.VMEM((_NI, _TM), jnp.float32),
            pltpu.VMEM((_NI, _TM), jnp.int32),
        ],
        compiler_params=pltpu.CompilerParams(
            dimension_semantics=("arbitrary", "arbitrary")),
        interpret=_INTERPRET,
    )(codebook, nw, z.reshape(_NI, _D, _TM), nz)
    return out.reshape(_M)


_BL = 2
_TL = _BL * _T


def _loss_body(q_ref, z_ref, qt_ref, loss_ref, acc_ref):
    i = pl.program_id(0)

    @pl.when(i == 0)
    def _():
        acc_ref[0, 0] = 0.0

    tot = jnp.float32(0.0)
    for k in range(_BL):
        qt = jnp.transpose(q_ref[pl.ds(k * _T, _T), :], (1, 0))
        qt_ref[k] = qt
        dlt = qt - z_ref[k]
        tot += jnp.sum(dlt * dlt)
    acc_ref[0, 0] += tot

    @pl.when(i == pl.num_programs(0) - 1)
    def _():
        loss_ref[...] = jnp.full((1, 1), acc_ref[0, 0] * (0.25 / (_M * _D)),
                                 jnp.float32)


def _loss_and_qout(q, z):
    nblk = _M // _TL
    qt, loss = pl.pallas_call(
        _loss_body,
        grid=(nblk,),
        in_specs=[
            pl.BlockSpec((_TL, _D), lambda i: (i, 0)),
            pl.BlockSpec((_BL, _D, _T), lambda i: (i, 0, 0)),
        ],
        out_specs=[
            pl.BlockSpec((_BL, _D, _T), lambda i: (i, 0, 0)),
            pl.BlockSpec((1, 1), lambda i: (0, 0)),
        ],
        out_shape=[
            jax.ShapeDtypeStruct((_B, _D, _T), jnp.float32),
            jax.ShapeDtypeStruct((1, 1), jnp.float32),
        ],
        scratch_shapes=[pltpu.SMEM((1, 1), jnp.float32)],
        interpret=_INTERPRET,
    )(q, z)
    return qt, loss[0, 0]


def _sc_gather(codebook, codes):
    info = plsc.get_sparse_core_info()
    nc, ns = info.num_cores, info.num_subcores
    nw = nc * ns
    b_per_w = _M // nw
    chunk = 128
    n_chunks = b_per_w // chunk
    mesh = plsc.VectorSubcoreMesh(core_axis_name="c", subcore_axis_name="s")

    @functools.partial(
        pl.kernel,
        mesh=mesh,
        out_type=jax.ShapeDtypeStruct((_M, _D), jnp.float32),
        scratch_types=[
            pltpu.VMEM((b_per_w,), jnp.int32),
            pltpu.VMEM((chunk, _D), jnp.float32),
            pltpu.VMEM((chunk, _D), jnp.float32),
            pltpu.SemaphoreType.DMA,
            pltpu.SemaphoreType.DMA,
        ],
    )
    def k(cb_hbm, idx_hbm, out_hbm, idx_v, r0, r1, s0, s1):
        wid = lax.axis_index("s") * nc + lax.axis_index("c")
        base = wid * b_per_w
        pltpu.sync_copy(idx_hbm.at[pl.ds(base, b_per_w)], idx_v)
        bufs, sems = (r0, r1), (s0, s1)
        copies = []
        for g in range(n_chunks):
            copies.append(pltpu.async_copy(
                cb_hbm.at[idx_v.at[pl.ds(g * chunk, chunk)]],
                bufs[g % 2], sems[g % 2]))
            if g >= 1:
                copies[g - 1].wait()
                pltpu.sync_copy(bufs[(g - 1) % 2],
                                out_hbm.at[pl.ds(base + (g - 1) * chunk,
                                                 chunk)])
        copies[-1].wait()
        pltpu.sync_copy(bufs[(n_chunks - 1) % 2],
                        out_hbm.at[pl.ds(base + (n_chunks - 1) * chunk,
                                         chunk)])

    return k(codebook, codes)


def kernel(z, codebook):
    b, d, t = z.shape
    zp = jnp.transpose(z, (0, 2, 1)).reshape(-1, d)
    nz = jnp.linalg.norm(zp, axis=-1)
    nw = jnp.linalg.norm(codebook, axis=-1, keepdims=True)
    codes = _argmin_codes(codebook, nw, z, nz.reshape(_NI, 1, _TM))
    q = _sc_gather(codebook, codes)
    q_out, loss = _loss_and_qout(q, z)
    return q_out, loss, codes.reshape(b, t)


# speedup vs baseline: 2.6696x; 1.0995x over previous
import functools

import jax
import jax.numpy as jnp
from jax import lax
from jax.experimental import pallas as pl
from jax.experimental.pallas import tpu as pltpu
from jax.experimental.pallas import tpu_sc as plsc

_INTERPRET = False

_B, _D, _T = 16, 256, 1024
_M = _B * _T
_K = 8192

_TM = 1024
_TN = 4096
_NI = _M // _TM
_NJ = _K // _TN

_NSUB = 8


def _argmin_body(cb_ref, nw_ref, z_ref, nz_ref, codes_ref,
                 wn_ref, best_d_ref, best_i_ref):
    j = pl.program_id(0)
    i = pl.program_id(1)

    @pl.when(i == 0)
    def _():
        wn_ref[...] = cb_ref[...] / jnp.maximum(nw_ref[...], 1e-6)

    znt = z_ref[0] / jnp.maximum(nz_ref[0], 1e-6)

    big = jnp.int32(2**31 - 1)
    rows_per = _TN // _NSUB
    acc_d = jnp.full((8, _TM), jnp.inf, jnp.float32)
    acc_g = jnp.full((8, _TM), big, jnp.int32)
    for c in range(_NSUB):
        s_c = jnp.dot(wn_ref[pl.ds(c * rows_per, rows_per), :], znt,
                      preferred_element_type=jnp.float32)
        for r in range(rows_per // 8):
            dblk = 1.0 - s_c[r * 8:(r + 1) * 8, :]
            g = jnp.full((8, _TM), c * rows_per // 8 + r, jnp.int32)
            take = dblk < acc_d
            acc_d = jnp.where(take, dblk, acc_d)
            acc_g = jnp.where(take, g, acc_g)
    sub = lax.broadcasted_iota(jnp.int32, (8, _TM), 0)
    acc_i = acc_g * 8 + sub + j * _TN
    m = jnp.min(acc_d, axis=0)
    idx = jnp.min(jnp.where(acc_d == m[None, :], acc_i, big), axis=0)

    @pl.when(j == 0)
    def _():
        best_d_ref[pl.ds(i, 1), :] = m[None, :]
        best_i_ref[pl.ds(i, 1), :] = idx[None, :]

    @pl.when(j > 0)
    def _():
        take = m[None, :] < best_d_ref[pl.ds(i, 1), :]
        best_d_ref[pl.ds(i, 1), :] = jnp.where(
            take, m[None, :], best_d_ref[pl.ds(i, 1), :])
        best_i_ref[pl.ds(i, 1), :] = jnp.where(
            take, idx[None, :], best_i_ref[pl.ds(i, 1), :])

    codes_ref[0, 0, :] = best_i_ref[pl.ds(i, 1), :][0, :]


def _argmin_codes(codebook, nw, z, nz):
    out = pl.pallas_call(
        _argmin_body,
        grid=(_NJ, _NI),
        in_specs=[
            pl.BlockSpec((_TN, _D), lambda j, i: (j, 0)),
            pl.BlockSpec((_TN, 1), lambda j, i: (j, 0)),
            pl.BlockSpec((1, _D, _TM), lambda j, i: (i, 0, 0)),
            pl.BlockSpec((1, 1, _TM), lambda j, i: (i, 0, 0)),
        ],
        out_specs=pl.BlockSpec((1, 1, _TM), lambda j, i: (i, 0, 0)),
        out_shape=jax.ShapeDtypeStruct((_NI, 1, _TM), jnp.int32),
        scratch_shapes=[
            pltpu.VMEM((_TN, _D), jnp.float32),
            pltpu.VMEM((_NI, _TM), jnp.float32),
            pltpu.VMEM((_NI, _TM), jnp.int32),
        ],
        compiler_params=pltpu.CompilerParams(
            dimension_semantics=("arbitrary", "arbitrary")),
        interpret=_INTERPRET,
    )(codebook, nw, z.reshape(_NI, _D, _TM), nz)
    return out.reshape(_M)


_BL = 4
_TL = _BL * _T


def _loss_body(q_ref, z_ref, qt_ref, loss_ref, acc_ref):
    i = pl.program_id(0)

    @pl.when(i == 0)
    def _():
        acc_ref[0, 0] = 0.0

    tot = jnp.float32(0.0)
    for k in range(_BL):
        qt = jnp.transpose(q_ref[pl.ds(k * _T, _T), :], (1, 0))
        qt_ref[k] = qt
        dlt = qt - z_ref[k]
        tot += jnp.sum(dlt * dlt)
    acc_ref[0, 0] += tot

    @pl.when(i == pl.num_programs(0) - 1)
    def _():
        loss_ref[...] = jnp.full((1, 1), acc_ref[0, 0] * (0.25 / (_M * _D)),
                                 jnp.float32)


def _loss_and_qout(q, z):
    nblk = _M // _TL
    qt, loss = pl.pallas_call(
        _loss_body,
        grid=(nblk,),
        in_specs=[
            pl.BlockSpec((_TL, _D), lambda i: (i, 0)),
            pl.BlockSpec((_BL, _D, _T), lambda i: (i, 0, 0)),
        ],
        out_specs=[
            pl.BlockSpec((_BL, _D, _T), lambda i: (i, 0, 0)),
            pl.BlockSpec((1, 1), lambda i: (0, 0)),
        ],
        out_shape=[
            jax.ShapeDtypeStruct((_B, _D, _T), jnp.float32),
            jax.ShapeDtypeStruct((1, 1), jnp.float32),
        ],
        scratch_shapes=[pltpu.SMEM((1, 1), jnp.float32)],
        interpret=_INTERPRET,
    )(q, z)
    return qt, loss[0, 0]


def _sc_gather(codebook, codes):
    info = plsc.get_sparse_core_info()
    nc, ns = info.num_cores, info.num_subcores
    nw = nc * ns
    b_per_w = _M // nw
    chunk = 128
    n_chunks = b_per_w // chunk
    mesh = plsc.VectorSubcoreMesh(core_axis_name="c", subcore_axis_name="s")

    @functools.partial(
        pl.kernel,
        mesh=mesh,
        out_type=jax.ShapeDtypeStruct((_M, _D), jnp.float32),
        scratch_types=[
            pltpu.VMEM((b_per_w,), jnp.int32),
            pltpu.VMEM((chunk, _D), jnp.float32),
            pltpu.VMEM((chunk, _D), jnp.float32),
            pltpu.SemaphoreType.DMA,
            pltpu.SemaphoreType.DMA,
        ],
    )
    def k(cb_hbm, idx_hbm, out_hbm, idx_v, r0, r1, s0, s1):
        wid = lax.axis_index("s") * nc + lax.axis_index("c")
        base = wid * b_per_w
        pltpu.sync_copy(idx_hbm.at[pl.ds(base, b_per_w)], idx_v)
        bufs, sems = (r0, r1), (s0, s1)
        copies = []
        for g in range(n_chunks):
            copies.append(pltpu.async_copy(
                cb_hbm.at[idx_v.at[pl.ds(g * chunk, chunk)]],
                bufs[g % 2], sems[g % 2]))
            if g >= 1:
                copies[g - 1].wait()
                pltpu.sync_copy(bufs[(g - 1) % 2],
                                out_hbm.at[pl.ds(base + (g - 1) * chunk,
                                                 chunk)])
        copies[-1].wait()
        pltpu.sync_copy(bufs[(n_chunks - 1) % 2],
                        out_hbm.at[pl.ds(base + (n_chunks - 1) * chunk,
                                         chunk)])

    return k(codebook, codes)


def kernel(z, codebook):
    b, d, t = z.shape
    zp = jnp.transpose(z, (0, 2, 1)).reshape(-1, d)
    nz = jnp.linalg.norm(zp, axis=-1)
    nw = jnp.linalg.norm(codebook, axis=-1, keepdims=True)
    codes = _argmin_codes(codebook, nw, z, nz.reshape(_NI, 1, _TM))
    q = _sc_gather(codebook, codes)
    q_out, loss = _loss_and_qout(q, z)
    return q_out, loss, codes.reshape(b, t)


# speedup vs baseline: 2.7514x; 1.0307x over previous
import functools

import jax
import jax.numpy as jnp
from jax import lax
from jax.experimental import pallas as pl
from jax.experimental.pallas import tpu as pltpu
from jax.experimental.pallas import tpu_sc as plsc

_INTERPRET = False

_B, _D, _T = 16, 256, 1024
_M = _B * _T
_K = 8192

_TM = 1024
_TN = 8192
_NI = _M // _TM
_NJ = _K // _TN

_NSUB = 16


def _argmin_body(cb_ref, nw_ref, z_ref, nz_ref, codes_ref,
                 wn_ref, best_d_ref, best_i_ref):
    j = pl.program_id(0)
    i = pl.program_id(1)

    @pl.when(i == 0)
    def _():
        wn_ref[...] = cb_ref[...] / jnp.maximum(nw_ref[...], 1e-6)

    znt = z_ref[0] / jnp.maximum(nz_ref[0], 1e-6)

    big = jnp.int32(2**31 - 1)
    rows_per = _TN // _NSUB
    acc_d = jnp.full((8, _TM), jnp.inf, jnp.float32)
    acc_g = jnp.full((8, _TM), big, jnp.int32)
    for c in range(_NSUB):
        s_c = jnp.dot(wn_ref[pl.ds(c * rows_per, rows_per), :], znt,
                      preferred_element_type=jnp.float32)
        for r in range(rows_per // 8):
            dblk = 1.0 - s_c[r * 8:(r + 1) * 8, :]
            g = jnp.full((8, _TM), c * rows_per // 8 + r, jnp.int32)
            take = dblk < acc_d
            acc_d = jnp.where(take, dblk, acc_d)
            acc_g = jnp.where(take, g, acc_g)
    sub = lax.broadcasted_iota(jnp.int32, (8, _TM), 0)
    acc_i = acc_g * 8 + sub + j * _TN
    m = jnp.min(acc_d, axis=0)
    idx = jnp.min(jnp.where(acc_d == m[None, :], acc_i, big), axis=0)

    @pl.when(j == 0)
    def _():
        best_d_ref[pl.ds(i, 1), :] = m[None, :]
        best_i_ref[pl.ds(i, 1), :] = idx[None, :]

    @pl.when(j > 0)
    def _():
        take = m[None, :] < best_d_ref[pl.ds(i, 1), :]
        best_d_ref[pl.ds(i, 1), :] = jnp.where(
            take, m[None, :], best_d_ref[pl.ds(i, 1), :])
        best_i_ref[pl.ds(i, 1), :] = jnp.where(
            take, idx[None, :], best_i_ref[pl.ds(i, 1), :])

    codes_ref[0, 0, :] = best_i_ref[pl.ds(i, 1), :][0, :]


def _argmin_codes(codebook, nw, z, nz):
    out = pl.pallas_call(
        _argmin_body,
        grid=(_NJ, _NI),
        in_specs=[
            pl.BlockSpec((_TN, _D), lambda j, i: (j, 0)),
            pl.BlockSpec((_TN, 1), lambda j, i: (j, 0)),
            pl.BlockSpec((1, _D, _TM), lambda j, i: (i, 0, 0)),
            pl.BlockSpec((1, 1, _TM), lambda j, i: (i, 0, 0)),
        ],
        out_specs=pl.BlockSpec((1, 1, _TM), lambda j, i: (i, 0, 0)),
        out_shape=jax.ShapeDtypeStruct((_NI, 1, _TM), jnp.int32),
        scratch_shapes=[
            pltpu.VMEM((_TN, _D), jnp.float32),
            pltpu.VMEM((_NI, _TM), jnp.float32),
            pltpu.VMEM((_NI, _TM), jnp.int32),
        ],
        compiler_params=pltpu.CompilerParams(
            dimension_semantics=("arbitrary", "arbitrary")),
        interpret=_INTERPRET,
    )(codebook, nw, z.reshape(_NI, _D, _TM), nz)
    return out.reshape(_M)


_BL = 4
_TL = _BL * _T


def _loss_body(q_ref, z_ref, qt_ref, loss_ref, acc_ref):
    i = pl.program_id(0)

    @pl.when(i == 0)
    def _():
        acc_ref[0, 0] = 0.0

    tot = jnp.float32(0.0)
    for k in range(_BL):
        qt = jnp.transpose(q_ref[pl.ds(k * _T, _T), :], (1, 0))
        qt_ref[k] = qt
        dlt = qt - z_ref[k]
        tot += jnp.sum(dlt * dlt)
    acc_ref[0, 0] += tot

    @pl.when(i == pl.num_programs(0) - 1)
    def _():
        loss_ref[...] = jnp.full((1, 1), acc_ref[0, 0] * (0.25 / (_M * _D)),
                                 jnp.float32)


def _loss_and_qout(q, z):
    nblk = _M // _TL
    qt, loss = pl.pallas_call(
        _loss_body,
        grid=(nblk,),
        in_specs=[
            pl.BlockSpec((_TL, _D), lambda i: (i, 0)),
            pl.BlockSpec((_BL, _D, _T), lambda i: (i, 0, 0)),
        ],
        out_specs=[
            pl.BlockSpec((_BL, _D, _T), lambda i: (i, 0, 0)),
            pl.BlockSpec((1, 1), lambda i: (0, 0)),
        ],
        out_shape=[
            jax.ShapeDtypeStruct((_B, _D, _T), jnp.float32),
            jax.ShapeDtypeStruct((1, 1), jnp.float32),
        ],
        scratch_shapes=[pltpu.SMEM((1, 1), jnp.float32)],
        interpret=_INTERPRET,
    )(q, z)
    return qt, loss[0, 0]


def _sc_gather(codebook, codes):
    info = plsc.get_sparse_core_info()
    nc, ns = info.num_cores, info.num_subcores
    nw = nc * ns
    b_per_w = _M // nw
    chunk = 128
    n_chunks = b_per_w // chunk
    mesh = plsc.VectorSubcoreMesh(core_axis_name="c", subcore_axis_name="s")

    @functools.partial(
        pl.kernel,
        mesh=mesh,
        out_type=jax.ShapeDtypeStruct((_M, _D), jnp.float32),
        scratch_types=[
            pltpu.VMEM((b_per_w,), jnp.int32),
            pltpu.VMEM((chunk, _D), jnp.float32),
            pltpu.VMEM((chunk, _D), jnp.float32),
            pltpu.SemaphoreType.DMA,
            pltpu.SemaphoreType.DMA,
        ],
    )
    def k(cb_hbm, idx_hbm, out_hbm, idx_v, r0, r1, s0, s1):
        wid = lax.axis_index("s") * nc + lax.axis_index("c")
        base = wid * b_per_w
        pltpu.sync_copy(idx_hbm.at[pl.ds(base, b_per_w)], idx_v)
        bufs, sems = (r0, r1), (s0, s1)
        copies = []
        for g in range(n_chunks):
            copies.append(pltpu.async_copy(
                cb_hbm.at[idx_v.at[pl.ds(g * chunk, chunk)]],
                bufs[g % 2], sems[g % 2]))
            if g >= 1:
                copies[g - 1].wait()
                pltpu.sync_copy(bufs[(g - 1) % 2],
                                out_hbm.at[pl.ds(base + (g - 1) * chunk,
                                                 chunk)])
        copies[-1].wait()
        pltpu.sync_copy(bufs[(n_chunks - 1) % 2],
                        out_hbm.at[pl.ds(base + (n_chunks - 1) * chunk,
                                         chunk)])

    return k(codebook, codes)


def kernel(z, codebook):
    b, d, t = z.shape
    zp = jnp.transpose(z, (0, 2, 1)).reshape(-1, d)
    nz = jnp.linalg.norm(zp, axis=-1)
    nw = jnp.linalg.norm(codebook, axis=-1, keepdims=True)
    codes = _argmin_codes(codebook, nw, z, nz.reshape(_NI, 1, _TM))
    q = _sc_gather(codebook, codes)
    q_out, loss = _loss_and_qout(q, z)
    return q_out, loss, codes.reshape(b, t)
